# baseline (speedup 1.0000x reference)
import functools

import jax
import jax.numpy as jnp
from jax import lax
from jax.experimental import pallas as pl
from jax.experimental.pallas import tpu as pltpu
from jax.experimental.pallas import tpu_sc as plsc

_N = 10000
_E = 320000
_D_IN = 128
_H1 = 8
_C1 = 8
_NCLS = 40
_NEG = 0.2

_NW = 32
_CH = 128
_NCHG = _E // _CH
_KBASE = _NCHG // _NW
_KREM = _NCHG % _NW


def _tc1_body(x_ref, w1_ref, a1s_ref, a1d_ref, htab_ref, dtab_ref):
    r = x_ref.shape[0]
    h = jnp.dot(x_ref[...], w1_ref[...], preferred_element_type=jnp.float32)
    a_s = jnp.dot(h, a1s_ref[...], preferred_element_type=jnp.float32)
    a_d = jnp.dot(h, a1d_ref[...], preferred_element_type=jnp.float32)
    z8 = jnp.zeros((r, 8), jnp.float32)
    htab_ref[...] = jnp.concatenate([h, a_s, z8], axis=1)
    dtab_ref[...] = jnp.concatenate([a_d, z8], axis=1)


def _tc2_body(p_ref, b1_ref, w2_ref, a2s_ref, a2d_ref, gtab_ref, d2tab_ref):
    r = p_ref.shape[1]
    acc = p_ref[0] + p_ref[1]
    cols = []
    for hh in range(_H1):
        num = acc[:, hh * _C1:(hh + 1) * _C1]
        den = acc[:, 64 + hh:65 + hh]
        cols.append(num / (den + 1e-16))
    h1 = jnp.concatenate(cols, axis=1) + b1_ref[...]
    h1 = jnp.where(h1 > 0, h1, jnp.exp(h1) - 1.0)
    g2 = jnp.dot(h1, w2_ref[...], preferred_element_type=jnp.float32)
    s2 = jnp.dot(g2, a2s_ref[...], preferred_element_type=jnp.float32)
    d2 = jnp.dot(g2, a2d_ref[...], preferred_element_type=jnp.float32)
    z1 = jnp.zeros((r, 1), jnp.float32)
    gtab_ref[...] = jnp.concatenate(
        [g2, z1, s2, jnp.zeros((r, 6), jnp.float32)], axis=1)
    d2tab_ref[...] = jnp.concatenate(
        [d2, jnp.zeros((r, 15), jnp.float32)], axis=1)


def _tc3_body(p_ref, b2_ref, o_ref):
    acc = p_ref[0] + p_ref[1]
    o = acc[:, :_NCLS] / (acc[:, _NCLS:_NCLS + 1] + 1e-16) + b2_ref[...]
    m = jnp.max(o, axis=1, keepdims=True)
    ls = m + jnp.log(jnp.sum(jnp.exp(o - m), axis=1, keepdims=True))
    o_ref[...] = o - ls


def _issue_idx(src_h, dst_h, base, sv, dv, sem):
    pltpu.async_copy(src_h.at[pl.ds(base, _CH)], sv, sem)
    pltpu.async_copy(dst_h.at[pl.ds(base, _CH)], dv, sem)


def _wait_idx(src_h, dst_h, sv, dv, sem):
    pltpu.make_async_copy(src_h.at[pl.ds(0, _CH)], sv, sem).wait()
    pltpu.make_async_copy(dst_h.at[pl.ds(0, _CH)], dv, sem).wait()


def _snapshot_idx(dst_v, sdst):
    for k in range(_CH // 16):
        sdst[pl.ds(16 * k, 16)] = dst_v[pl.ds(16 * k, 16)]


def _sc_edges1_body(src_h, dst_h, htab_h, dtab_h, zeros_h, out_h,
                    acc, src_v, dst_v, sdst, hrow, drow, eexp, msg,
                    sem_i, sem_g, sem_s):
    cid = lax.axis_index("c")
    sid = lax.axis_index("s")
    wid = sid * 2 + cid

    @pl.when(sid == 0)
    def _():
        pltpu.sync_copy(zeros_h, acc)
    plsc.subcore_barrier()

    iota = lax.iota(jnp.int32, 16)
    rowoff = iota >> 3
    coloff = iota & 7
    nk = _KBASE + jnp.where(wid < _KREM, 1, 0)

    def kbase(k):
        return (wid + _NW * k) * _CH

    def issue_gathers(b):
        pltpu.async_copy(htab_h.at[src_v[b]], hrow[b], sem_g[b])
        pltpu.async_copy(dtab_h.at[dst_v[b]], drow[b], sem_g[b])

    def wait_gathers(b):
        pltpu.make_async_copy(htab_h.at[src_v[b]], hrow[b], sem_g[b]).wait()
        pltpu.make_async_copy(dtab_h.at[dst_v[b]], drow[b], sem_g[b]).wait()

    def compute(b):
        _snapshot_idx(dst_v[b], sdst[b])

        def eblk(k, c):
            i0 = k * 2
            s = plsc.load_gather(hrow[b], [i0 + rowoff, 64 + coloff])
            d = plsc.load_gather(drow[b], [i0 + rowoff, coloff])
            e = s + d
            e = jnp.where(e >= 0, e, _NEG * e)
            eexp[b][pl.ds(k * 16, 16)] = jnp.exp(e)
            return c
        lax.fori_loop(0, _CH // 2, eblk, 0, unroll=4)

        def medge(i, c):
            for q in range(4):
                ev = plsc.load_gather(eexp[b], [i * 8 + 2 * q + rowoff])
                msg[b][i, pl.ds(16 * q, 16)] = hrow[b][i, pl.ds(16 * q, 16)] * ev
            return c
        lax.fori_loop(0, _CH, medge, 0, unroll=4)

        def cblk(k, c):
            i0 = k * 2
            ev = eexp[b][pl.ds(k * 16, 16)]
            plsc.store_scatter(msg[b], [i0 + rowoff, 64 + coloff], ev)
            return c
        lax.fori_loop(0, _CH // 2, cblk, 0, unroll=4)

    def issue_scatter(b):
        pltpu.async_copy(msg[b], acc.at[sdst[b]], sem_s[b], add=True)

    def wait_scatter(b):
        pltpu.make_async_copy(msg[b], acc.at[sdst[b]], sem_s[b]).wait()

    _issue_idx(src_h, dst_h, kbase(0), src_v[0], dst_v[0], sem_i[0])
    _issue_idx(src_h, dst_h, kbase(1), src_v[1], dst_v[1], sem_i[1])
    _wait_idx(src_h, dst_h, src_v[0], dst_v[0], sem_i[0])
    issue_gathers(0)

    def outer(it, carry):
        gbase = it * 2
        for b in range(2):
            g = gbase + b
            ob = 1 - b

            @pl.when(g < nk)
            def _():
                @pl.when(g + 1 < nk)
                def _():
                    _wait_idx(src_h, dst_h, src_v[ob], dst_v[ob], sem_i[ob])
                    issue_gathers(ob)
                wait_gathers(b)

                @pl.when(g + 2 < nk)
                def _():
                    _issue_idx(src_h, dst_h, kbase(g + 2),
                               src_v[b], dst_v[b], sem_i[b])

                @pl.when(g >= 2)
                def _():
                    wait_scatter(b)
                compute(b)
                issue_scatter(b)
        return carry
    lax.fori_loop(0, (nk + 1) // 2, outer, 0)
    wait_scatter(0)
    wait_scatter(1)

    plsc.subcore_barrier()

    @pl.when(sid == 0)
    def _():
        pltpu.sync_copy(acc, out_h.at[cid])


def _sc_edges2_body(src_h, dst_h, gtab_h, d2tab_h, zeros_h, out_h,
                    acc, src_v, dst_v, sdst, grow, drow, eexp, msg,
                    sem_i, sem_g, sem_s):
    cid = lax.axis_index("c")
    sid = lax.axis_index("s")
    wid = sid * 2 + cid

    @pl.when(sid == 0)
    def _():
        pltpu.sync_copy(zeros_h, acc)
    plsc.subcore_barrier()

    iota = lax.iota(jnp.int32, 16)
    col41 = jnp.full((16,), 41, jnp.int32)
    col0 = jnp.zeros((16,), jnp.int32)
    one = jnp.ones((16,), jnp.float32)
    nk = _KBASE + jnp.where(wid < _KREM, 1, 0)

    def kbase(k):
        return (wid + _NW * k) * _CH

    def issue_gathers(b):
        pltpu.async_copy(gtab_h.at[src_v[b]], grow[b], sem_g[b])
        pltpu.async_copy(d2tab_h.at[dst_v[b]], drow[b], sem_g[b])

    def wait_gathers(b):
        pltpu.make_async_copy(gtab_h.at[src_v[b]], grow[b], sem_g[b]).wait()
        pltpu.make_async_copy(d2tab_h.at[dst_v[b]], drow[b], sem_g[b]).wait()

    def compute(b):
        _snapshot_idx(dst_v[b], sdst[b])

        def eblk(k, c):
            i0 = k * 16
            s = plsc.load_gather(grow[b], [i0 + iota, col41])
            d = plsc.load_gather(drow[b], [i0 + iota, col0])
            e = s + d
            e = jnp.where(e >= 0, e, _NEG * e)
            eexp[b][pl.ds(i0, 16)] = jnp.exp(e)
            return c
        lax.fori_loop(0, _CH // 16, eblk, 0, unroll=4)

        def medge(i, c):
            ev = plsc.load_gather(eexp[b], [jnp.broadcast_to(i, (16,))])
            for q in range(3):
                rv = grow[b][i, pl.ds(16 * q, 16)]
                if q == 2:
                    rv = jnp.where(iota == 8, one, rv)
                msg[b][i, pl.ds(16 * q, 16)] = rv * ev
            return c
        lax.fori_loop(0, _CH, medge, 0, unroll=4)

    def issue_scatter(b):
        pltpu.async_copy(msg[b], acc.at[sdst[b]], sem_s[b], add=True)

    def wait_scatter(b):
        pltpu.make_async_copy(msg[b], acc.at[sdst[b]], sem_s[b]).wait()

    _issue_idx(src_h, dst_h, kbase(0), src_v[0], dst_v[0], sem_i[0])
    _issue_idx(src_h, dst_h, kbase(1), src_v[1], dst_v[1], sem_i[1])
    _wait_idx(src_h, dst_h, src_v[0], dst_v[0], sem_i[0])
    issue_gathers(0)

    def outer(it, carry):
        gbase = it * 2
        for b in range(2):
            g = gbase + b
            ob = 1 - b

            @pl.when(g < nk)
            def _():
                @pl.when(g + 1 < nk)
                def _():
                    _wait_idx(src_h, dst_h, src_v[ob], dst_v[ob], sem_i[ob])
                    issue_gathers(ob)
                wait_gathers(b)

                @pl.when(g + 2 < nk)
                def _():
                    _issue_idx(src_h, dst_h, kbase(g + 2),
                               src_v[b], dst_v[b], sem_i[b])

                @pl.when(g >= 2)
                def _():
                    wait_scatter(b)
                compute(b)
                issue_scatter(b)
        return carry
    lax.fori_loop(0, (nk + 1) // 2, outer, 0)
    wait_scatter(0)
    wait_scatter(1)

    plsc.subcore_barrier()

    @pl.when(sid == 0)
    def _():
        pltpu.sync_copy(acc, out_h.at[cid])


_SC_MESH = plsc.VectorSubcoreMesh(core_axis_name="c", subcore_axis_name="s")

_sc_edges1 = pl.kernel(
    _sc_edges1_body,
    out_type=jax.ShapeDtypeStruct((2, _N, 72), jnp.float32),
    mesh=_SC_MESH,
    compiler_params=pltpu.CompilerParams(needs_layout_passes=False, use_tc_tiling_on_sc=False),
    scratch_types=[
        pltpu.VMEM_SHARED((_N, 72), jnp.float32),
        [pltpu.VMEM((_CH,), jnp.int32)] * 2,
        [pltpu.VMEM((_CH,), jnp.int32)] * 2,
        [pltpu.VMEM((_CH,), jnp.int32)] * 2,
        [pltpu.VMEM((_CH, 80), jnp.float32)] * 2,
        [pltpu.VMEM((_CH, 16), jnp.float32)] * 2,
        [pltpu.VMEM((_CH * 8,), jnp.float32)] * 2,
        [pltpu.VMEM((_CH, 72), jnp.float32)] * 2,
        [pltpu.SemaphoreType.DMA] * 2,
        [pltpu.SemaphoreType.DMA] * 2,
        [pltpu.SemaphoreType.DMA] * 2,
    ],
)

_sc_edges2 = pl.kernel(
    _sc_edges2_body,
    out_type=jax.ShapeDtypeStruct((2, _N, 48), jnp.float32),
    mesh=_SC_MESH,
    compiler_params=pltpu.CompilerParams(needs_layout_passes=False, use_tc_tiling_on_sc=False),
    scratch_types=[
        pltpu.VMEM_SHARED((_N, 48), jnp.float32),
        [pltpu.VMEM((_CH,), jnp.int32)] * 2,
        [pltpu.VMEM((_CH,), jnp.int32)] * 2,
        [pltpu.VMEM((_CH,), jnp.int32)] * 2,
        [pltpu.VMEM((_CH, 48), jnp.float32)] * 2,
        [pltpu.VMEM((_CH, 16), jnp.float32)] * 2,
        [pltpu.VMEM((_CH,), jnp.float32)] * 2,
        [pltpu.VMEM((_CH, 48), jnp.float32)] * 2,
        [pltpu.SemaphoreType.DMA] * 2,
        [pltpu.SemaphoreType.DMA] * 2,
        [pltpu.SemaphoreType.DMA] * 2,
    ],
)

_R = 1000


def _tc1(x, w1, a1s, a1d):
    return pl.pallas_call(
        _tc1_body,
        grid=(_N // _R,),
        in_specs=[
            pl.BlockSpec((_R, _D_IN), lambda i: (i, 0)),
            pl.BlockSpec((_D_IN, 64), lambda i: (0, 0)),
            pl.BlockSpec((64, 8), lambda i: (0, 0)),
            pl.BlockSpec((64, 8), lambda i: (0, 0)),
        ],
        out_specs=[
            pl.BlockSpec((_R, 80), lambda i: (i, 0)),
            pl.BlockSpec((_R, 16), lambda i: (i, 0)),
        ],
        out_shape=[
            jax.ShapeDtypeStruct((_N, 80), jnp.float32),
            jax.ShapeDtypeStruct((_N, 16), jnp.float32),
        ],
    )(x, w1, a1s, a1d)


def _tc2(p1, b1, w2, a2s, a2d):
    return pl.pallas_call(
        _tc2_body,
        grid=(_N // _R,),
        in_specs=[
            pl.BlockSpec((2, _R, 72), lambda i: (0, i, 0)),
            pl.BlockSpec((1, 64), lambda i: (0, 0)),
            pl.BlockSpec((64, _NCLS), lambda i: (0, 0)),
            pl.BlockSpec((_NCLS, 1), lambda i: (0, 0)),
            pl.BlockSpec((_NCLS, 1), lambda i: (0, 0)),
        ],
        out_specs=[
            pl.BlockSpec((_R, 48), lambda i: (i, 0)),
            pl.BlockSpec((_R, 16), lambda i: (i, 0)),
        ],
        out_shape=[
            jax.ShapeDtypeStruct((_N, 48), jnp.float32),
            jax.ShapeDtypeStruct((_N, 16), jnp.float32),
        ],
    )(p1, b1, w2, a2s, a2d)


def _tc3(p2, b2):
    return pl.pallas_call(
        _tc3_body,
        grid=(_N // _R,),
        in_specs=[
            pl.BlockSpec((2, _R, 48), lambda i: (0, i, 0)),
            pl.BlockSpec((1, _NCLS), lambda i: (0, 0)),
        ],
        out_specs=pl.BlockSpec((_R, _NCLS), lambda i: (i, 0)),
        out_shape=jax.ShapeDtypeStruct((_N, _NCLS), jnp.float32),
    )(p2, b2)


@jax.jit
def kernel(x, edge_index, W1, a1_src, a1_dst, b1, W2, a2_src, a2_dst, b2):
    src = edge_index[0].astype(jnp.int32)
    dst = edge_index[1].astype(jnp.int32)
    eye = jnp.eye(_H1, dtype=jnp.float32)
    a1s = (a1_src[:, :, None] * eye[:, None, :]).reshape(_H1 * _C1, _H1)
    a1d = (a1_dst[:, :, None] * eye[:, None, :]).reshape(_H1 * _C1, _H1)

    htab, dtab = _tc1(x, W1, a1s, a1d)
    p1 = _sc_edges1(src, dst, htab, dtab, jnp.zeros((_N, 72), jnp.float32))
    gtab, d2tab = _tc2(p1, b1.reshape(1, 64), W2,
                       a2_src.reshape(_NCLS, 1), a2_dst.reshape(_NCLS, 1))
    p2 = _sc_edges2(src, dst, gtab, d2tab, jnp.zeros((_N, 48), jnp.float32))
    return _tc3(p2, b2.reshape(1, _NCLS))


# speedup vs baseline: 1.2508x; 1.2508x over previous
import functools

import jax
import jax.numpy as jnp
from jax import lax
from jax.experimental import pallas as pl
from jax.experimental.pallas import tpu as pltpu
from jax.experimental.pallas import tpu_sc as plsc

_N = 10000
_E = 320000
_D_IN = 128
_H1 = 8
_C1 = 8
_NCLS = 40
_NEG = 0.2

_NW = 32
_CH = 128
_NCHG = _E // _CH
_KBASE = _NCHG // _NW
_KREM = _NCHG % _NW


def _tc1_body(x_ref, w1_ref, a1s_ref, a1d_ref, htab_ref, dtab_ref):
    r = x_ref.shape[0]
    h = jnp.dot(x_ref[...], w1_ref[...], preferred_element_type=jnp.float32)
    a_s = jnp.dot(h, a1s_ref[...], preferred_element_type=jnp.float32)
    a_d = jnp.dot(h, a1d_ref[...], preferred_element_type=jnp.float32)
    z8 = jnp.zeros((r, 8), jnp.float32)
    htab_ref[...] = jnp.concatenate([h, a_s, z8], axis=1)
    dtab_ref[...] = jnp.concatenate([a_d, z8], axis=1)


def _tc2_body(p_ref, b1_ref, w2_ref, a2s_ref, a2d_ref, gtab_ref, d2tab_ref):
    r = p_ref.shape[1]
    acc = p_ref[0] + p_ref[1]
    cols = []
    for hh in range(_H1):
        num = acc[:, hh * _C1:(hh + 1) * _C1]
        den = acc[:, 64 + hh:65 + hh]
        cols.append(num / (den + 1e-16))
    h1 = jnp.concatenate(cols, axis=1) + b1_ref[...]
    h1 = jnp.where(h1 > 0, h1, jnp.exp(h1) - 1.0)
    g2 = jnp.dot(h1, w2_ref[...], preferred_element_type=jnp.float32)
    s2 = jnp.dot(g2, a2s_ref[...], preferred_element_type=jnp.float32)
    d2 = jnp.dot(g2, a2d_ref[...], preferred_element_type=jnp.float32)
    z1 = jnp.zeros((r, 1), jnp.float32)
    gtab_ref[...] = jnp.concatenate(
        [g2, z1, s2, jnp.zeros((r, 6), jnp.float32)], axis=1)
    d2tab_ref[...] = jnp.concatenate(
        [d2, jnp.zeros((r, 15), jnp.float32)], axis=1)


def _tc3_body(p_ref, b2_ref, o_ref):
    acc = p_ref[0] + p_ref[1]
    o = acc[:, :_NCLS] / (acc[:, _NCLS:_NCLS + 1] + 1e-16) + b2_ref[...]
    m = jnp.max(o, axis=1, keepdims=True)
    ls = m + jnp.log(jnp.sum(jnp.exp(o - m), axis=1, keepdims=True))
    o_ref[...] = o - ls


def _issue_idx(src_h, dst_h, base, sv, dv, sem):
    pltpu.async_copy(src_h.at[pl.ds(base, _CH)], sv, sem)
    pltpu.async_copy(dst_h.at[pl.ds(base, _CH)], dv, sem)


def _wait_idx(src_h, dst_h, sv, dv, sem):
    pltpu.make_async_copy(src_h.at[pl.ds(0, _CH)], sv, sem).wait()
    pltpu.make_async_copy(dst_h.at[pl.ds(0, _CH)], dv, sem).wait()


def _snapshot_idx(dst_v, sdst):
    for k in range(_CH // 16):
        sdst[pl.ds(16 * k, 16)] = dst_v[pl.ds(16 * k, 16)]


_DYN_DNUMS = lax.GatherDimensionNumbers(
    offset_dims=(), collapsed_slice_dims=(0,), start_index_map=(0,))


def _dyng(vec, idx):
    return lax.gather(vec, idx[:, None], _DYN_DNUMS, (1,),
                      mode=lax.GatherScatterMode.PROMISE_IN_BOUNDS)


def _sc_edges1_body(src_h, dst_h, htab_h, dtab_h, zeros_h, out_h,
                    acc, src_v, dst_v, sdst, hrow, drow, eexp, msg,
                    sem_i, sem_g, sem_s):
    cid = lax.axis_index("c")
    sid = lax.axis_index("s")
    wid = sid * 2 + cid

    @pl.when(sid == 0)
    def _():
        pltpu.sync_copy(zeros_h, acc)
    plsc.subcore_barrier()

    iota = lax.iota(jnp.int32, 16)
    rowoff = iota >> 3
    coloff = iota & 7
    nk = _KBASE + jnp.where(wid < _KREM, 1, 0)

    def kbase(k):
        return (wid + _NW * k) * _CH

    def issue_gathers(b):
        pltpu.async_copy(htab_h.at[src_v[b]], hrow[b], sem_g[b])
        pltpu.async_copy(dtab_h.at[dst_v[b]], drow[b], sem_g[b])

    def wait_gathers(b):
        pltpu.make_async_copy(htab_h.at[src_v[b]], hrow[b], sem_g[b]).wait()
        pltpu.make_async_copy(dtab_h.at[dst_v[b]], drow[b], sem_g[b]).wait()

    def compute(b):
        _snapshot_idx(dst_v[b], sdst[b])

        def eblk(k, c):
            i0 = k * 2
            s = plsc.load_gather(hrow[b], [i0 + rowoff, 64 + coloff])
            d = plsc.load_gather(drow[b], [i0 + rowoff, coloff])
            e = s + d
            e = jnp.where(e >= 0, e, _NEG * e)
            eexp[b][pl.ds(k * 16, 16)] = jnp.exp(e)
            return c
        lax.fori_loop(0, _CH // 2, eblk, 0, unroll=4)

        def mpair(p, c):
            i = p * 2
            epair = eexp[b][pl.ds(p * 16, 16)]
            plsc.store_scatter(msg[b], [i + rowoff, 64 + coloff], epair)
            for sub in range(2):
                for q in range(4):
                    ev = _dyng(epair, 8 * sub + 2 * q + rowoff)
                    msg[b][i + sub, pl.ds(16 * q, 16)] = (
                        hrow[b][i + sub, pl.ds(16 * q, 16)] * ev)
            return c
        lax.fori_loop(0, _CH // 2, mpair, 0, unroll=2)

    def issue_scatter(b):
        pltpu.async_copy(msg[b], acc.at[sdst[b]], sem_s[b], add=True)

    def wait_scatter(b):
        pltpu.make_async_copy(msg[b], acc.at[sdst[b]], sem_s[b]).wait()

    _issue_idx(src_h, dst_h, kbase(0), src_v[0], dst_v[0], sem_i[0])
    _issue_idx(src_h, dst_h, kbase(1), src_v[1], dst_v[1], sem_i[1])
    _wait_idx(src_h, dst_h, src_v[0], dst_v[0], sem_i[0])
    issue_gathers(0)

    def outer(it, carry):
        gbase = it * 2
        for b in range(2):
            g = gbase + b
            ob = 1 - b

            @pl.when(g < nk)
            def _():
                @pl.when(g + 1 < nk)
                def _():
                    _wait_idx(src_h, dst_h, src_v[ob], dst_v[ob], sem_i[ob])
                    issue_gathers(ob)
                wait_gathers(b)

                @pl.when(g + 2 < nk)
                def _():
                    _issue_idx(src_h, dst_h, kbase(g + 2),
                               src_v[b], dst_v[b], sem_i[b])

                @pl.when(g >= 2)
                def _():
                    wait_scatter(b)
                compute(b)
                issue_scatter(b)
        return carry
    lax.fori_loop(0, (nk + 1) // 2, outer, 0)
    wait_scatter(0)
    wait_scatter(1)

    plsc.subcore_barrier()

    @pl.when(sid == 0)
    def _():
        pltpu.sync_copy(acc, out_h.at[cid])


def _sc_edges2_body(src_h, dst_h, gtab_h, d2tab_h, zeros_h, out_h,
                    acc, src_v, dst_v, sdst, grow, drow, eexp, msg,
                    sem_i, sem_g, sem_s):
    cid = lax.axis_index("c")
    sid = lax.axis_index("s")
    wid = sid * 2 + cid

    @pl.when(sid == 0)
    def _():
        pltpu.sync_copy(zeros_h, acc)
    plsc.subcore_barrier()

    iota = lax.iota(jnp.int32, 16)
    col41 = jnp.full((16,), 41, jnp.int32)
    col0 = jnp.zeros((16,), jnp.int32)
    one = jnp.ones((16,), jnp.float32)
    nk = _KBASE + jnp.where(wid < _KREM, 1, 0)

    def kbase(k):
        return (wid + _NW * k) * _CH

    def issue_gathers(b):
        pltpu.async_copy(gtab_h.at[src_v[b]], grow[b], sem_g[b])
        pltpu.async_copy(d2tab_h.at[dst_v[b]], drow[b], sem_g[b])

    def wait_gathers(b):
        pltpu.make_async_copy(gtab_h.at[src_v[b]], grow[b], sem_g[b]).wait()
        pltpu.make_async_copy(d2tab_h.at[dst_v[b]], drow[b], sem_g[b]).wait()

    def compute(b):
        _snapshot_idx(dst_v[b], sdst[b])

        def eblk(k, c):
            i0 = k * 16
            s = plsc.load_gather(grow[b], [i0 + iota, col41])
            d = plsc.load_gather(drow[b], [i0 + iota, col0])
            e = s + d
            e = jnp.where(e >= 0, e, _NEG * e)
            eexp[b][pl.ds(i0, 16)] = jnp.exp(e)
            return c
        lax.fori_loop(0, _CH // 16, eblk, 0, unroll=4)

        splats = [jnp.full((16,), jj, jnp.int32) for jj in range(16)]

        def mgrp(gg, c):
            i0 = gg * 16
            e16 = eexp[b][pl.ds(i0, 16)]
            for jj in range(16):
                ev = _dyng(e16, splats[jj])
                i = i0 + jj
                for q in range(3):
                    rv = grow[b][i, pl.ds(16 * q, 16)]
                    if q == 2:
                        rv = jnp.where(iota == 8, one, rv)
                    msg[b][i, pl.ds(16 * q, 16)] = rv * ev
            return c
        lax.fori_loop(0, _CH // 16, mgrp, 0)

    def issue_scatter(b):
        pltpu.async_copy(msg[b], acc.at[sdst[b]], sem_s[b], add=True)

    def wait_scatter(b):
        pltpu.make_async_copy(msg[b], acc.at[sdst[b]], sem_s[b]).wait()

    _issue_idx(src_h, dst_h, kbase(0), src_v[0], dst_v[0], sem_i[0])
    _issue_idx(src_h, dst_h, kbase(1), src_v[1], dst_v[1], sem_i[1])
    _wait_idx(src_h, dst_h, src_v[0], dst_v[0], sem_i[0])
    issue_gathers(0)

    def outer(it, carry):
        gbase = it * 2
        for b in range(2):
            g = gbase + b
            ob = 1 - b

            @pl.when(g < nk)
            def _():
                @pl.when(g + 1 < nk)
                def _():
                    _wait_idx(src_h, dst_h, src_v[ob], dst_v[ob], sem_i[ob])
                    issue_gathers(ob)
                wait_gathers(b)

                @pl.when(g + 2 < nk)
                def _():
                    _issue_idx(src_h, dst_h, kbase(g + 2),
                               src_v[b], dst_v[b], sem_i[b])

                @pl.when(g >= 2)
                def _():
                    wait_scatter(b)
                compute(b)
                issue_scatter(b)
        return carry
    lax.fori_loop(0, (nk + 1) // 2, outer, 0)
    wait_scatter(0)
    wait_scatter(1)

    plsc.subcore_barrier()

    @pl.when(sid == 0)
    def _():
        pltpu.sync_copy(acc, out_h.at[cid])


_SC_MESH = plsc.VectorSubcoreMesh(core_axis_name="c", subcore_axis_name="s")

_sc_edges1 = pl.kernel(
    _sc_edges1_body,
    out_type=jax.ShapeDtypeStruct((2, _N, 72), jnp.float32),
    mesh=_SC_MESH,
    compiler_params=pltpu.CompilerParams(needs_layout_passes=False, use_tc_tiling_on_sc=False),
    scratch_types=[
        pltpu.VMEM_SHARED((_N, 72), jnp.float32),
        [pltpu.VMEM((_CH,), jnp.int32)] * 2,
        [pltpu.VMEM((_CH,), jnp.int32)] * 2,
        [pltpu.VMEM((_CH,), jnp.int32)] * 2,
        [pltpu.VMEM((_CH, 80), jnp.float32)] * 2,
        [pltpu.VMEM((_CH, 16), jnp.float32)] * 2,
        [pltpu.VMEM((_CH * 8,), jnp.float32)] * 2,
        [pltpu.VMEM((_CH, 72), jnp.float32)] * 2,
        [pltpu.SemaphoreType.DMA] * 2,
        [pltpu.SemaphoreType.DMA] * 2,
        [pltpu.SemaphoreType.DMA] * 2,
    ],
)

_sc_edges2 = pl.kernel(
    _sc_edges2_body,
    out_type=jax.ShapeDtypeStruct((2, _N, 48), jnp.float32),
    mesh=_SC_MESH,
    compiler_params=pltpu.CompilerParams(needs_layout_passes=False, use_tc_tiling_on_sc=False),
    scratch_types=[
        pltpu.VMEM_SHARED((_N, 48), jnp.float32),
        [pltpu.VMEM((_CH,), jnp.int32)] * 2,
        [pltpu.VMEM((_CH,), jnp.int32)] * 2,
        [pltpu.VMEM((_CH,), jnp.int32)] * 2,
        [pltpu.VMEM((_CH, 48), jnp.float32)] * 2,
        [pltpu.VMEM((_CH, 16), jnp.float32)] * 2,
        [pltpu.VMEM((_CH,), jnp.float32)] * 2,
        [pltpu.VMEM((_CH, 48), jnp.float32)] * 2,
        [pltpu.SemaphoreType.DMA] * 2,
        [pltpu.SemaphoreType.DMA] * 2,
        [pltpu.SemaphoreType.DMA] * 2,
    ],
)

_R = 1000


def _tc1(x, w1, a1s, a1d):
    return pl.pallas_call(
        _tc1_body,
        grid=(_N // _R,),
        in_specs=[
            pl.BlockSpec((_R, _D_IN), lambda i: (i, 0)),
            pl.BlockSpec((_D_IN, 64), lambda i: (0, 0)),
            pl.BlockSpec((64, 8), lambda i: (0, 0)),
            pl.BlockSpec((64, 8), lambda i: (0, 0)),
        ],
        out_specs=[
            pl.BlockSpec((_R, 80), lambda i: (i, 0)),
            pl.BlockSpec((_R, 16), lambda i: (i, 0)),
        ],
        out_shape=[
            jax.ShapeDtypeStruct((_N, 80), jnp.float32),
            jax.ShapeDtypeStruct((_N, 16), jnp.float32),
        ],
    )(x, w1, a1s, a1d)


def _tc2(p1, b1, w2, a2s, a2d):
    return pl.pallas_call(
        _tc2_body,
        grid=(_N // _R,),
        in_specs=[
            pl.BlockSpec((2, _R, 72), lambda i: (0, i, 0)),
            pl.BlockSpec((1, 64), lambda i: (0, 0)),
            pl.BlockSpec((64, _NCLS), lambda i: (0, 0)),
            pl.BlockSpec((_NCLS, 1), lambda i: (0, 0)),
            pl.BlockSpec((_NCLS, 1), lambda i: (0, 0)),
        ],
        out_specs=[
            pl.BlockSpec((_R, 48), lambda i: (i, 0)),
            pl.BlockSpec((_R, 16), lambda i: (i, 0)),
        ],
        out_shape=[
            jax.ShapeDtypeStruct((_N, 48), jnp.float32),
            jax.ShapeDtypeStruct((_N, 16), jnp.float32),
        ],
    )(p1, b1, w2, a2s, a2d)


def _tc3(p2, b2):
    return pl.pallas_call(
        _tc3_body,
        grid=(_N // _R,),
        in_specs=[
            pl.BlockSpec((2, _R, 48), lambda i: (0, i, 0)),
            pl.BlockSpec((1, _NCLS), lambda i: (0, 0)),
        ],
        out_specs=pl.BlockSpec((_R, _NCLS), lambda i: (i, 0)),
        out_shape=jax.ShapeDtypeStruct((_N, _NCLS), jnp.float32),
    )(p2, b2)


@jax.jit
def kernel(x, edge_index, W1, a1_src, a1_dst, b1, W2, a2_src, a2_dst, b2):
    src = edge_index[0].astype(jnp.int32)
    dst = edge_index[1].astype(jnp.int32)
    eye = jnp.eye(_H1, dtype=jnp.float32)
    a1s = (a1_src[:, :, None] * eye[:, None, :]).reshape(_H1 * _C1, _H1)
    a1d = (a1_dst[:, :, None] * eye[:, None, :]).reshape(_H1 * _C1, _H1)

    htab, dtab = _tc1(x, W1, a1s, a1d)
    p1 = _sc_edges1(src, dst, htab, dtab, jnp.zeros((_N, 72), jnp.float32))
    gtab, d2tab = _tc2(p1, b1.reshape(1, 64), W2,
                       a2_src.reshape(_NCLS, 1), a2_dst.reshape(_NCLS, 1))
    p2 = _sc_edges2(src, dst, gtab, d2tab, jnp.zeros((_N, 48), jnp.float32))
    return _tc3(p2, b2.reshape(1, _NCLS))
